# baseline (speedup 1.0000x reference)
import functools

import jax
import jax.numpy as jnp
from jax import lax
from jax.experimental import pallas as pl
from jax.experimental.pallas import tpu as pltpu
from jax.experimental.pallas import tpu_sc as plsc

D = 1024
ROWS = 16384
NW = 32
ROWS_PER_W = ROWS // NW
C = 4
NCHUNK = ROWS_PER_W // C
NBUF = 8
P = NBUF // 2
LANES = 16

_mesh = plsc.VectorSubcoreMesh(core_axis_name="c", subcore_axis_name="s")


@functools.partial(
    pl.kernel,
    mesh=_mesh,
    out_type=jax.ShapeDtypeStruct((ROWS, D), jnp.float32),
    scratch_types=[
        pltpu.VMEM((NCHUNK, C), jnp.int32),
        pltpu.VMEM((NBUF, C, D), jnp.float32),
        pltpu.VMEM((NBUF, C, D), jnp.float32),
    ] + [pltpu.SemaphoreType.DMA] * (2 * NBUF),
)
def _sc_kernel(x_hbm, idx_hbm, pe_hbm, out_hbm, idx_v, xbuf, pebuf, *sems):
    sem_in = sems[:NBUF]
    sem_out = sems[NBUF:]
    wid = lax.axis_index("s") * 2 + lax.axis_index("c")
    base = wid * ROWS_PER_W
    pltpu.sync_copy(idx_hbm.at[wid], idx_v)

    def start_in(c, b):
        off = base + c * C
        pltpu.async_copy(x_hbm.at[pl.ds(off, C)], xbuf.at[b], sem_in[b])
        pltpu.async_copy(pe_hbm.at[idx_v.at[c]], pebuf.at[b], sem_in[b])

    def wait_in(b):
        pltpu.make_async_copy(x_hbm.at[pl.ds(0, C)], xbuf.at[b], sem_in[b]).wait()
        pltpu.make_async_copy(x_hbm.at[pl.ds(0, C)], pebuf.at[b], sem_in[b]).wait()

    def wait_out(b):
        pltpu.make_async_copy(x_hbm.at[pl.ds(0, C)], xbuf.at[b], sem_out[b]).wait()

    for b in range(P):
        start_in(b, b)

    @pl.loop(0, NCHUNK, step=NBUF)
    def _outer(o):
        for b in range(NBUF):
            c = o + b
            wait_in(b)

            @pl.loop(0, C)
            def _row(j):
                for g in range(D // LANES):
                    sl = pl.ds(g * LANES, LANES)
                    plsc.addupdate(xbuf.at[b, j, sl], pebuf[b, j, sl])

            pltpu.async_copy(xbuf.at[b], out_hbm.at[pl.ds(base + c * C, C)],
                             sem_out[b])

            bp = (b + P) % NBUF

            @pl.when(jnp.logical_and(c + P >= NBUF, c + P < NCHUNK))
            def _():
                wait_out(bp)
                start_in(c + P, bp)

            @pl.when(c + P < NBUF)
            def _():
                start_in(c + P, bp)

    for b in range(NBUF):
        wait_out(b)


def kernel(x, pos_id_torch_pad, pe_table):
    xf = x.reshape(ROWS, D)
    idx = pos_id_torch_pad.astype(jnp.int32).reshape(NW, NCHUNK, C)
    out = _sc_kernel(xf, idx, pe_table)
    return out.reshape(x.shape)
